# Initial kernel scaffold; baseline (speedup 1.0000x reference)
import jax
import jax.numpy as jnp
from jax.experimental import pallas as pl


def kernel(x, edge_index, edge_weight, mask_nodes, token_nodes, noise_nodes, noise_src, mask_token, enc_W1, enc_b1, enc_W2, enc_b2, e2d_W, dec_W1, dec_b1, dec_W2, dec_b2):
    raise NotImplementedError("write your pallas kernel here")


# speedup vs baseline: 9.4656x; 9.4656x over previous
import functools

import jax
import jax.numpy as jnp
from jax import lax
from jax.experimental import pallas as pl
from jax.experimental.pallas import tpu as pltpu
from jax.experimental.pallas import tpu_sc as plsc

N = 10000
E = 320000
D = 128
NC = 2
NS = 16
NW = NC * NS
EPT = E // NW
EB = EPT // 16
ROWS_PER_TILE = N // NS

_SC_MESH = plsc.VectorSubcoreMesh(core_axis_name="c", subcore_axis_name="s")
_SC_PARAMS = pltpu.CompilerParams(use_tc_tiling_on_sc=False,
                                  needs_layout_passes=False)


def _strided_range(wid, nblocks, nworkers):
    return (nblocks - wid + nworkers - 1) // nworkers


def _sc_pre_body(x_hbm, tok2d_hbm, noise2d_hbm, nsrc2d_hbm, mask2d_hbm,
                 moff_hbm, mtok_hbm, col2d_hbm, ew2d_hbm,
                 outx_hbm, xm_hbm, degp_hbm,
                 buf_v, tokbuf_v, idx_v, ioff_v, colc_v, ewc_v, zb_v,
                 acc_deg, sem):
    c = lax.axis_index("c")
    s = lax.axis_index("s")

    n_tok_b = tok2d_hbm.shape[0]
    n_noise_b = noise2d_hbm.shape[0]
    n_mask_b = mask2d_hbm.shape[0]

    @pl.when(c == 0)
    def _core0():
        nb = N // 16
        cnt = _strided_range(s, nb, NS)

        def copy_body(i, _):
            b = s + i * NS
            pltpu.sync_copy(x_hbm.at[pl.ds(b * 16, 16)], buf_v)
            pltpu.sync_copy(buf_v, outx_hbm.at[pl.ds(b * 16, 16)])
            return 0

        lax.fori_loop(0, cnt, copy_body, 0)
        plsc.subcore_barrier()

        pltpu.sync_copy(mtok_hbm, tokbuf_v.at[pl.ds(0, 1)])
        for f in range(D // 16):
            v = tokbuf_v[0, pl.ds(f * 16, 16)]
            for r in range(1, 16):
                tokbuf_v[r, pl.ds(f * 16, 16)] = v

        tcnt = _strided_range(s, n_tok_b, NS)

        def tok_body(i, _):
            b = s + i * NS
            pltpu.sync_copy(tok2d_hbm.at[b], idx_v)
            pltpu.sync_copy(tokbuf_v, outx_hbm.at[idx_v])
            return 0

        lax.fori_loop(0, tcnt, tok_body, 0)

        ncnt = _strided_range(s, n_noise_b, NS)

        def noise_body(i, _):
            b = s + i * NS
            pltpu.sync_copy(nsrc2d_hbm.at[b], idx_v)
            pltpu.async_copy(x_hbm.at[idx_v], buf_v, sem).wait()
            pltpu.sync_copy(noise2d_hbm.at[b], idx_v)
            pltpu.sync_copy(buf_v, outx_hbm.at[idx_v])
            return 0

        lax.fori_loop(0, ncnt, noise_body, 0)

        mcnt = _strided_range(s, n_mask_b, NS)

        def mask_body(i, _):
            b = s + i * NS
            pltpu.sync_copy(mask2d_hbm.at[b], idx_v)
            pltpu.sync_copy(moff_hbm.at[b], ioff_v)
            pltpu.async_copy(x_hbm.at[idx_v], buf_v, sem).wait()
            off = pl.multiple_of(ioff_v[...][0], 8)
            pltpu.sync_copy(buf_v, xm_hbm.at[pl.ds(off, 16)])
            return 0

        lax.fori_loop(0, mcnt, mask_body, 0)

    @pl.when(c == 1)
    def _core1():
        for t in range(40):
            zb_v[pl.ds(t * 16, 16)] = jnp.zeros((16,), jnp.float32)
        pltpu.sync_copy(zb_v, acc_deg.at[pl.ds(s * 640, 640)])
        plsc.subcore_barrier()

        epb = E // NS // 16
        pltpu.sync_copy(col2d_hbm.at[s], colc_v)
        pltpu.sync_copy(ew2d_hbm.at[s], ewc_v)

        def deg_body(i, _):
            pltpu.sync_copy(ewc_v.at[i], acc_deg.at[colc_v.at[i]], add=True)
            return 0

        lax.fori_loop(0, epb, deg_body, 0)
        plsc.subcore_barrier()
        pltpu.sync_copy(acc_deg.at[pl.ds(s * 640, 640)],
                        degp_hbm.at[pl.ds(s * 640, 640)])


def _sc_pre(x, tok2d, noise2d, nsrc2d, mask2d, moff, mtok, col2d, ew2d,
            n_mask):
    return pl.kernel(
        _sc_pre_body,
        out_type=(
            jax.ShapeDtypeStruct((N, D), jnp.float32),
            jax.ShapeDtypeStruct((n_mask, D), jnp.float32),
            jax.ShapeDtypeStruct((10240,), jnp.float32),
        ),
        mesh=_SC_MESH,
        compiler_params=_SC_PARAMS,
        scratch_types=[
            pltpu.VMEM((16, D), jnp.float32),
            pltpu.VMEM((16, D), jnp.float32),
            pltpu.VMEM((16,), jnp.int32),
            pltpu.VMEM((16,), jnp.int32),
            pltpu.VMEM((E // NS // 16, 16), jnp.int32),
            pltpu.VMEM((E // NS // 16, 16), jnp.float32),
            pltpu.VMEM((640,), jnp.float32),
            pltpu.VMEM_SHARED((10240,), jnp.float32),
            pltpu.SemaphoreType.DMA,
        ],
    )(x, tok2d, noise2d, nsrc2d, mask2d, moff, mtok, col2d, ew2d)


def _sc_norm_body(row_hbm, col_hbm, ew_hbm, dinv_hbm, norm_hbm,
                  dinv_v, row_v, col_v, ew_v, nrm_v):
    c = lax.axis_index("c")
    s = lax.axis_index("s")
    wid = s * NC + c

    pltpu.sync_copy(dinv_hbm, dinv_v)
    pltpu.sync_copy(row_hbm.at[wid], row_v)
    pltpu.sync_copy(col_hbm.at[wid], col_v)
    pltpu.sync_copy(ew_hbm.at[wid], ew_v)

    def body(j, _):
        r16 = row_v[pl.ds(j * 16, 16)]
        c16 = col_v[pl.ds(j * 16, 16)]
        g1 = plsc.load_gather(dinv_v, [r16])
        g2 = plsc.load_gather(dinv_v, [c16])
        w16 = ew_v[pl.ds(j * 16, 16)]
        nrm_v[pl.ds(j * 16, 16)] = g1 * w16 * g2
        return 0

    lax.fori_loop(0, EB, body, 0)
    pltpu.sync_copy(nrm_v, norm_hbm.at[wid])


def _sc_norm(row, col, ew, dinv):
    return pl.kernel(
        _sc_norm_body,
        out_type=jax.ShapeDtypeStruct((NW, EPT), jnp.float32),
        mesh=_SC_MESH,
        compiler_params=_SC_PARAMS,
        scratch_types=[
            pltpu.VMEM((N,), jnp.float32),
            pltpu.VMEM((EPT,), jnp.int32),
            pltpu.VMEM((EPT,), jnp.int32),
            pltpu.VMEM((EPT,), jnp.float32),
            pltpu.VMEM((EPT,), jnp.float32),
        ],
    )(row, col, ew, dinv)


def _sc_msg_body(h_hbm, row_hbm, col2d_hbm, norm_hbm, zeros_hbm, p_hbm,
                 row_v, col_v, nrm_v, bufa_v, bufb_v,
                 acc, sema, semb):
    c = lax.axis_index("c")
    s = lax.axis_index("s")
    wid = s * NC + c

    @pl.when(s < NS - 1)
    def _():
        pltpu.sync_copy(zeros_hbm, acc.at[pl.ds(s * 640, 640)])

    @pl.when(s == NS - 1)
    def _():
        pltpu.sync_copy(zeros_hbm.at[pl.ds(0, 400)], acc.at[pl.ds(9600, 400)])

    plsc.subcore_barrier()

    pltpu.sync_copy(row_hbm.at[wid], row_v)
    pltpu.sync_copy(col2d_hbm.at[wid], col_v)
    pltpu.sync_copy(norm_hbm.at[wid], nrm_v)

    def scale(buf, j):
        nb = nrm_v[pl.ds(j * 16, 16)]
        for k in range(16):
            sv = jnp.full((16,), nb[k], jnp.float32)
            for f in range(D // 16):
                buf[k, pl.ds(f * 16, 16)] = buf[k, pl.ds(f * 16, 16)] * sv

    def start_gather(buf, sem, j):
        pltpu.async_copy(h_hbm.at[row_v.at[pl.ds(j * 16, 16)]], buf, sem)

    def wait_gather(buf, sem):
        pltpu.make_async_copy(h_hbm.at[pl.ds(0, 16)], buf, sem).wait()

    start_gather(bufa_v, sema, 0)

    def body(i, _):
        j0 = 2 * i
        start_gather(bufb_v, semb, j0 + 1)
        wait_gather(bufa_v, sema)
        scale(bufa_v, j0)
        pltpu.sync_copy(bufa_v, acc.at[col_v.at[j0]], add=True)

        @pl.when(j0 + 2 < EB)
        def _():
            start_gather(bufa_v, sema, j0 + 2)

        wait_gather(bufb_v, semb)
        scale(bufb_v, j0 + 1)
        pltpu.sync_copy(bufb_v, acc.at[col_v.at[j0 + 1]], add=True)
        return 0

    lax.fori_loop(0, EB // 2, body, 0)
    wait_gather(bufa_v, sema)
    scale(bufa_v, EB - 1)
    pltpu.sync_copy(bufa_v, acc.at[col_v.at[EB - 1]], add=True)

    plsc.subcore_barrier()

    @pl.when(s < NS - 1)
    def _():
        pltpu.sync_copy(acc.at[pl.ds(s * 640, 640)],
                        p_hbm.at[c, pl.ds(s * 640, 640)])

    @pl.when(s == NS - 1)
    def _():
        pltpu.sync_copy(acc.at[pl.ds(9600, 400)],
                        p_hbm.at[c, pl.ds(9600, 400)])


def _sc_msg(h, row, col2d, norm, zeros):
    return pl.kernel(
        _sc_msg_body,
        out_type=jax.ShapeDtypeStruct((NC, N, D), jnp.float32),
        mesh=_SC_MESH,
        compiler_params=_SC_PARAMS,
        scratch_types=[
            pltpu.VMEM((EPT,), jnp.int32),
            pltpu.VMEM((EB, 16), jnp.int32),
            pltpu.VMEM((EPT,), jnp.float32),
            pltpu.VMEM((16, D), jnp.float32),
            pltpu.VMEM((16, D), jnp.float32),
            pltpu.VMEM_SHARED((10240, D), jnp.float32),
            pltpu.SemaphoreType.DMA,
            pltpu.SemaphoreType.DMA,
        ],
    )(h, row, col2d, norm, zeros)


def _sc_gather_body(src_hbm, mask2d_hbm, moff_hbm, out_hbm, buf_v, idx_v,
                    ioff_v, sem):
    c = lax.axis_index("c")
    s = lax.axis_index("s")
    wid = s * NC + c
    nb = mask2d_hbm.shape[0]
    cnt = _strided_range(wid, nb, NW)

    def body(i, _):
        b = wid + i * NW
        pltpu.sync_copy(mask2d_hbm.at[b], idx_v)
        pltpu.sync_copy(moff_hbm.at[b], ioff_v)
        pltpu.async_copy(src_hbm.at[idx_v], buf_v, sem).wait()
        off = pl.multiple_of(ioff_v[...][0], 8)
        pltpu.sync_copy(buf_v, out_hbm.at[pl.ds(off, 16)])
        return 0

    lax.fori_loop(0, cnt, body, 0)


def _sc_gather(src, mask2d, moff, n_out):
    return pl.kernel(
        _sc_gather_body,
        out_type=jax.ShapeDtypeStruct((n_out, D), jnp.float32),
        mesh=_SC_MESH,
        compiler_params=_SC_PARAMS,
        scratch_types=[
            pltpu.VMEM((16, D), jnp.float32),
            pltpu.VMEM((16,), jnp.int32),
            pltpu.VMEM((16,), jnp.int32),
            pltpu.SemaphoreType.DMA,
        ],
    )(src, mask2d, moff)


BN = 1000


def _tc1_body(outx_ref, w_ref, degp_ref, h_ref, dinv_ref, dsq_ref):
    d = degp_ref[...] + 1.0
    dv = jax.lax.rsqrt(d)
    dinv_ref[...] = dv
    dsq_ref[...] = 1.0 / d
    h_ref[...] = jnp.dot(outx_ref[...], w_ref[...],
                         preferred_element_type=jnp.float32)


def _tc1(outx, W1, degp):
    return pl.pallas_call(
        _tc1_body,
        grid=(N // BN,),
        in_specs=[
            pl.BlockSpec((BN, D), lambda i: (i, 0)),
            pl.BlockSpec((D, D), lambda i: (0, 0)),
            pl.BlockSpec((BN, 1), lambda i: (i, 0)),
        ],
        out_specs=[
            pl.BlockSpec((BN, D), lambda i: (i, 0)),
            pl.BlockSpec((BN, 1), lambda i: (i, 0)),
            pl.BlockSpec((BN, 1), lambda i: (i, 0)),
        ],
        out_shape=[
            jax.ShapeDtypeStruct((N, D), jnp.float32),
            jax.ShapeDtypeStruct((N, 1), jnp.float32),
            jax.ShapeDtypeStruct((N, 1), jnp.float32),
        ],
    )(outx, W1, degp)


def _tc_combine_body(p_ref, h_ref, dsq_ref, b_ref, w_ref, z_ref, hn_ref, *,
                     do_relu, do_mm):
    z = p_ref[0] + p_ref[1] + dsq_ref[...] * h_ref[...] + b_ref[...]
    z_ref[...] = z
    if do_mm:
        a = jnp.maximum(z, 0.0) if do_relu else z
        hn_ref[...] = jnp.dot(a, w_ref[...], preferred_element_type=jnp.float32)


def _tc_combine(p, h, dsq, b, W, do_relu, do_mm):
    body = functools.partial(_tc_combine_body, do_relu=do_relu, do_mm=do_mm)
    return pl.pallas_call(
        body,
        grid=(N // BN,),
        in_specs=[
            pl.BlockSpec((NC, BN, D), lambda i: (0, i, 0)),
            pl.BlockSpec((BN, D), lambda i: (i, 0)),
            pl.BlockSpec((BN, 1), lambda i: (i, 0)),
            pl.BlockSpec((1, D), lambda i: (0, 0)),
            pl.BlockSpec((D, D), lambda i: (0, 0)),
        ],
        out_specs=[
            pl.BlockSpec((BN, D), lambda i: (i, 0)),
            pl.BlockSpec((BN, D), lambda i: (i, 0)),
        ],
        out_shape=[
            jax.ShapeDtypeStruct((N, D), jnp.float32),
            jax.ShapeDtypeStruct((N, D), jnp.float32),
        ],
    )(p, h, dsq, b, W)


def _tc_wf_body(a_ref, b_ref, o_ref):
    o_ref[...] = lax.dot_general(a_ref[...], b_ref[...],
                                 (((0,), (0,)), ((), ())),
                                 preferred_element_type=jnp.float32)


def _tc_wf(e2d_W, dec_W1):
    return pl.pallas_call(
        _tc_wf_body,
        out_shape=jax.ShapeDtypeStruct((D, D), jnp.float32),
    )(e2d_W, dec_W1)


def _tc_loss_body(xm_ref, dxm_ref, o_ref, *, n_mask):
    i = pl.program_id(0)

    @pl.when(i == 0)
    def _():
        o_ref[...] = jnp.zeros_like(o_ref)

    x = xm_ref[...]
    y = dxm_ref[...]
    nx = jnp.maximum(jnp.sqrt(jnp.sum(x * x, axis=1, keepdims=True)), 1e-12)
    ny = jnp.maximum(jnp.sqrt(jnp.sum(y * y, axis=1, keepdims=True)), 1e-12)
    dot = jnp.sum(x * y, axis=1, keepdims=True)
    r = 1.0 - dot / (nx * ny)
    o_ref[...] += jnp.sum(r * r).reshape(1, 1) * (1.0 / n_mask)


def _tc_loss(xm, dxm):
    n_mask = xm.shape[0]
    bm = 1000
    body = functools.partial(_tc_loss_body, n_mask=n_mask)
    return pl.pallas_call(
        body,
        grid=(n_mask // bm,),
        in_specs=[
            pl.BlockSpec((bm, D), lambda i: (i, 0)),
            pl.BlockSpec((bm, D), lambda i: (i, 0)),
        ],
        out_specs=pl.BlockSpec((1, 1), lambda i: (0, 0)),
        out_shape=jax.ShapeDtypeStruct((1, 1), jnp.float32),
    )(xm, dxm)


def _pad_reshape16(idx):
    n = idx.shape[0]
    nb = (n + 15) // 16
    pad = nb * 16 - n
    if pad:
        idx = jnp.concatenate([idx, idx[:pad]])
    return idx.reshape(nb, 16)


def _overlap_tail16(idx):
    import numpy as np
    n = idx.shape[0]
    nb = (n + 15) // 16
    offs = np.minimum(np.arange(nb) * 16, n - 16).astype(np.int32)
    rows = [idx[o:o + 16] for o in offs]
    blocks = jnp.stack(rows)
    offs2d = jnp.asarray(np.broadcast_to(offs[:, None], (nb, 16)).copy())
    return blocks, offs2d


def kernel(x, edge_index, edge_weight, mask_nodes, token_nodes, noise_nodes,
           noise_src, mask_token, enc_W1, enc_b1, enc_W2, enc_b2, e2d_W,
           dec_W1, dec_b1, dec_W2, dec_b2):
    row = edge_index[0].astype(jnp.int32)
    col = edge_index[1].astype(jnp.int32)
    ew = edge_weight
    col_deg = col.reshape(NS, E // NS // 16, 16)
    ew_deg = ew.reshape(NS, E // NS // 16, 16)
    row_w = row.reshape(NW, EPT)
    col_w = col.reshape(NW, EPT)
    ew_w = ew.reshape(NW, EPT)
    col_b = col.reshape(NW, EB, 16)
    zeros640 = jnp.zeros((640, D), jnp.float32)

    tok2d = _pad_reshape16(token_nodes)
    noise2d = _pad_reshape16(noise_nodes)
    nsrc2d = _pad_reshape16(noise_src)
    mask2d, moff = _overlap_tail16(mask_nodes)
    n_mask = mask_nodes.shape[0]

    outx, xm, degp = _sc_pre(x, tok2d, noise2d, nsrc2d, mask2d, moff,
                             mask_token, col_deg, ew_deg, n_mask)
    degp_n = degp[:N].reshape(N, 1)

    h1, dinv, dsq = _tc1(outx, enc_W1, degp_n)
    norm = _sc_norm(row_w, col_w, ew_w, dinv.reshape(N))

    bias = lambda b: b.reshape(1, D)
    wf = _tc_wf(e2d_W, dec_W1)

    p1 = _sc_msg(h1, row_w, col_b, norm, zeros640)
    _, h2 = _tc_combine(p1, h1, dsq, bias(enc_b1), enc_W2, True, True)
    p2 = _sc_msg(h2, row_w, col_b, norm, zeros640)
    enc_x, h3 = _tc_combine(p2, h2, dsq, bias(enc_b2), wf, False, True)
    p3 = _sc_msg(h3, row_w, col_b, norm, zeros640)
    _, h4 = _tc_combine(p3, h3, dsq, bias(dec_b1), dec_W2, True, True)
    p4 = _sc_msg(h4, row_w, col_b, norm, zeros640)
    dec_x, _ = _tc_combine(p4, h4, dsq, bias(dec_b2), dec_W2, False, False)

    dxm = _sc_gather(dec_x, mask2d, moff, n_mask)
    loss = _tc_loss(xm, dxm)[0, 0]
    return (enc_x, loss)


# speedup vs baseline: 16.6985x; 1.7641x over previous
import functools

import jax
import jax.numpy as jnp
from jax import lax
from jax.experimental import pallas as pl
from jax.experimental.pallas import tpu as pltpu
from jax.experimental.pallas import tpu_sc as plsc

N = 10000
E = 320000
D = 128
NC = 2
NS = 16
NW = NC * NS
EPT = E // NW
EB = EPT // 16
BLK = 80
NBLK = EPT // BLK
ROWS_PER_TILE = N // NS

_SC_MESH = plsc.VectorSubcoreMesh(core_axis_name="c", subcore_axis_name="s")
_SC_PARAMS = pltpu.CompilerParams(use_tc_tiling_on_sc=False,
                                  needs_layout_passes=False)


def _strided_range(wid, nblocks, nworkers):
    return (nblocks - wid + nworkers - 1) // nworkers


def _sc_pre_body(x_hbm, tok2d_hbm, noise2d_hbm, nsrc2d_hbm, mask2d_hbm,
                 moff_hbm, mtok_hbm, col2d_hbm, ew2d_hbm,
                 outx_hbm, xm_hbm, degp_hbm,
                 buf_v, tokbuf_v, idx_v, ioff_v, colc_v, ewc_v, zb_v,
                 acc_deg, sem):
    c = lax.axis_index("c")
    s = lax.axis_index("s")

    n_tok_b = tok2d_hbm.shape[0]
    n_noise_b = noise2d_hbm.shape[0]
    n_mask_b = mask2d_hbm.shape[0]

    @pl.when(c == 0)
    def _core0():
        nb = N // 16
        cnt = _strided_range(s, nb, NS)

        def copy_body(i, _):
            b = s + i * NS
            pltpu.sync_copy(x_hbm.at[pl.ds(b * 16, 16)], buf_v)
            pltpu.sync_copy(buf_v, outx_hbm.at[pl.ds(b * 16, 16)])
            return 0

        lax.fori_loop(0, cnt, copy_body, 0)
        plsc.subcore_barrier()

        pltpu.sync_copy(mtok_hbm, tokbuf_v.at[pl.ds(0, 1)])
        for f in range(D // 16):
            v = tokbuf_v[0, pl.ds(f * 16, 16)]
            for r in range(1, 16):
                tokbuf_v[r, pl.ds(f * 16, 16)] = v

        tcnt = _strided_range(s, n_tok_b, NS)

        def tok_body(i, _):
            b = s + i * NS
            pltpu.sync_copy(tok2d_hbm.at[b], idx_v)
            pltpu.sync_copy(tokbuf_v, outx_hbm.at[idx_v])
            return 0

        lax.fori_loop(0, tcnt, tok_body, 0)

        ncnt = _strided_range(s, n_noise_b, NS)

        def noise_body(i, _):
            b = s + i * NS
            pltpu.sync_copy(nsrc2d_hbm.at[b], idx_v)
            pltpu.async_copy(x_hbm.at[idx_v], buf_v, sem).wait()
            pltpu.sync_copy(noise2d_hbm.at[b], idx_v)
            pltpu.sync_copy(buf_v, outx_hbm.at[idx_v])
            return 0

        lax.fori_loop(0, ncnt, noise_body, 0)

        mcnt = _strided_range(s, n_mask_b, NS)

        def mask_body(i, _):
            b = s + i * NS
            pltpu.sync_copy(mask2d_hbm.at[b], idx_v)
            pltpu.sync_copy(moff_hbm.at[b], ioff_v)
            pltpu.async_copy(x_hbm.at[idx_v], buf_v, sem).wait()
            off = pl.multiple_of(ioff_v[...][0], 8)
            pltpu.sync_copy(buf_v, xm_hbm.at[pl.ds(off, 16)])
            return 0

        lax.fori_loop(0, mcnt, mask_body, 0)

    @pl.when(c == 1)
    def _core1():
        for t in range(40):
            zb_v[pl.ds(t * 16, 16)] = jnp.zeros((16,), jnp.float32)
        pltpu.sync_copy(zb_v, acc_deg.at[pl.ds(s * 640, 640)])
        plsc.subcore_barrier()

        epb = E // NS // 16
        pltpu.sync_copy(col2d_hbm.at[s], colc_v)
        pltpu.sync_copy(ew2d_hbm.at[s], ewc_v)

        def deg_body(i, _):
            pltpu.sync_copy(ewc_v.at[i], acc_deg.at[colc_v.at[i]], add=True)
            return 0

        lax.fori_loop(0, epb, deg_body, 0)
        plsc.subcore_barrier()
        pltpu.sync_copy(acc_deg.at[pl.ds(s * 640, 640)],
                        degp_hbm.at[pl.ds(s * 640, 640)])


def _sc_pre(x, tok2d, noise2d, nsrc2d, mask2d, moff, mtok, col2d, ew2d,
            n_mask):
    return pl.kernel(
        _sc_pre_body,
        out_type=(
            jax.ShapeDtypeStruct((N, D), jnp.float32),
            jax.ShapeDtypeStruct((n_mask, D), jnp.float32),
            jax.ShapeDtypeStruct((10240,), jnp.float32),
        ),
        mesh=_SC_MESH,
        compiler_params=_SC_PARAMS,
        scratch_types=[
            pltpu.VMEM((16, D), jnp.float32),
            pltpu.VMEM((16, D), jnp.float32),
            pltpu.VMEM((16,), jnp.int32),
            pltpu.VMEM((16,), jnp.int32),
            pltpu.VMEM((E // NS // 16, 16), jnp.int32),
            pltpu.VMEM((E // NS // 16, 16), jnp.float32),
            pltpu.VMEM((640,), jnp.float32),
            pltpu.VMEM_SHARED((10240,), jnp.float32),
            pltpu.SemaphoreType.DMA,
        ],
    )(x, tok2d, noise2d, nsrc2d, mask2d, moff, mtok, col2d, ew2d)


def _sc_norm_body(row_hbm, col_hbm, ew_hbm, dinv_hbm, norm_hbm,
                  dinv_v, row_v, col_v, ew_v, nrm_v):
    c = lax.axis_index("c")
    s = lax.axis_index("s")
    wid = s * NC + c

    pltpu.sync_copy(dinv_hbm, dinv_v)
    pltpu.sync_copy(row_hbm.at[wid], row_v)
    pltpu.sync_copy(col_hbm.at[wid], col_v)
    pltpu.sync_copy(ew_hbm.at[wid], ew_v)

    def body(j, _):
        r16 = row_v[pl.ds(j * 16, 16)]
        c16 = col_v[pl.ds(j * 16, 16)]
        g1 = plsc.load_gather(dinv_v, [r16])
        g2 = plsc.load_gather(dinv_v, [c16])
        w16 = ew_v[pl.ds(j * 16, 16)]
        nrm_v[pl.ds(j * 16, 16)] = g1 * w16 * g2
        return 0

    lax.fori_loop(0, EB, body, 0)
    pltpu.sync_copy(nrm_v, norm_hbm.at[wid])


def _sc_norm(row, col, ew, dinv):
    return pl.kernel(
        _sc_norm_body,
        out_type=jax.ShapeDtypeStruct((NW, EPT), jnp.float32),
        mesh=_SC_MESH,
        compiler_params=_SC_PARAMS,
        scratch_types=[
            pltpu.VMEM((N,), jnp.float32),
            pltpu.VMEM((EPT,), jnp.int32),
            pltpu.VMEM((EPT,), jnp.int32),
            pltpu.VMEM((EPT,), jnp.float32),
            pltpu.VMEM((EPT,), jnp.float32),
        ],
    )(row, col, ew, dinv)


def _sc_msg_body(h_hbm, row_hbm, col2d_hbm, norm_hbm, zeros_hbm, p_hbm,
                 row_v, col_v, nrm_v, buf0, buf1,
                 acc, sg0, sg1):
    c = lax.axis_index("c")
    s = lax.axis_index("s")
    wid = s * NC + c
    bufs = (buf0, buf1)
    sgs = (sg0, sg1)

    @pl.when(s < NS - 1)
    def _():
        pltpu.sync_copy(zeros_hbm, acc.at[pl.ds(s * 640, 640)])

    @pl.when(s == NS - 1)
    def _():
        pltpu.sync_copy(zeros_hbm.at[pl.ds(0, 400)], acc.at[pl.ds(9600, 400)])

    pltpu.sync_copy(row_hbm.at[wid], row_v)
    pltpu.sync_copy(col2d_hbm.at[wid], col_v)
    pltpu.sync_copy(norm_hbm.at[wid], nrm_v)
    plsc.subcore_barrier()

    def start_g(b, j):
        pltpu.async_copy(h_hbm.at[row_v.at[pl.ds(j * BLK, BLK)]], bufs[b],
                         sgs[b])

    def wait_g(b):
        pltpu.make_async_copy(h_hbm.at[pl.ds(0, BLK)], bufs[b], sgs[b]).wait()

    def start_s(b, j):
        pltpu.async_copy(bufs[b], acc.at[col_v.at[j]], sss[b], add=True)

    def wait_s(b):
        pltpu.make_async_copy(h_hbm.at[pl.ds(0, BLK)], bufs[b], sss[b]).wait()

    def scale(b, j):
        buf = bufs[b]

        def grp(g, _):
            nb = nrm_v[pl.ds(j * BLK + g * 16, 16)]
            for k in range(16):
                sv = jnp.full((16,), nb[k], jnp.float32)
                r = g * 16 + k
                for f in range(D // 16):
                    buf[r, pl.ds(f * 16, 16)] = buf[r, pl.ds(f * 16, 16)] * sv
            return 0

        lax.fori_loop(0, BLK // 16, grp, 0)

    for b in range(2):
        start_g(b, b)

    def body(i, _):
        for b in range(2):
            j = 2 * i + b
            wait_g(b)
            scale(b, j)
            pltpu.sync_copy(bufs[b], acc.at[col_v.at[j]], add=True)

            @pl.when(j + 2 < NBLK)
            def _():
                start_g(b, j + 2)
        return 0

    lax.fori_loop(0, NBLK // 2, body, 0)
    for r in range(NBLK % 2):
        j = (NBLK // 2) * 2 + r
        wait_g(r)
        scale(r, j)
        pltpu.sync_copy(bufs[r], acc.at[col_v.at[j]], add=True)

    plsc.subcore_barrier()

    @pl.when(s < NS - 1)
    def _():
        pltpu.sync_copy(acc.at[pl.ds(s * 640, 640)],
                        p_hbm.at[c, pl.ds(s * 640, 640)])

    @pl.when(s == NS - 1)
    def _():
        pltpu.sync_copy(acc.at[pl.ds(9600, 400)],
                        p_hbm.at[c, pl.ds(9600, 400)])


def _sc_msg(h, row, col2d, norm, zeros):
    return pl.kernel(
        _sc_msg_body,
        out_type=jax.ShapeDtypeStruct((NC, N, D), jnp.float32),
        mesh=_SC_MESH,
        compiler_params=_SC_PARAMS,
        scratch_types=[
            pltpu.VMEM((EPT,), jnp.int32),
            pltpu.VMEM((NBLK, BLK), jnp.int32),
            pltpu.VMEM((EPT,), jnp.float32),
            pltpu.VMEM((BLK, D), jnp.float32),
            pltpu.VMEM((BLK, D), jnp.float32),
            pltpu.VMEM_SHARED((N, D), jnp.float32),
            pltpu.SemaphoreType.DMA,
            pltpu.SemaphoreType.DMA,
        ],
    )(h, row, col2d, norm, zeros)


def _sc_gather_body(src_hbm, mask2d_hbm, moff_hbm, out_hbm, buf_v, idx_v,
                    ioff_v, sem):
    c = lax.axis_index("c")
    s = lax.axis_index("s")
    wid = s * NC + c
    nb = mask2d_hbm.shape[0]
    cnt = _strided_range(wid, nb, NW)

    def body(i, _):
        b = wid + i * NW
        pltpu.sync_copy(mask2d_hbm.at[b], idx_v)
        pltpu.sync_copy(moff_hbm.at[b], ioff_v)
        pltpu.async_copy(src_hbm.at[idx_v], buf_v, sem).wait()
        off = pl.multiple_of(ioff_v[...][0], 8)
        pltpu.sync_copy(buf_v, out_hbm.at[pl.ds(off, 16)])
        return 0

    lax.fori_loop(0, cnt, body, 0)


def _sc_gather(src, mask2d, moff, n_out):
    return pl.kernel(
        _sc_gather_body,
        out_type=jax.ShapeDtypeStruct((n_out, D), jnp.float32),
        mesh=_SC_MESH,
        compiler_params=_SC_PARAMS,
        scratch_types=[
            pltpu.VMEM((16, D), jnp.float32),
            pltpu.VMEM((16,), jnp.int32),
            pltpu.VMEM((16,), jnp.int32),
            pltpu.SemaphoreType.DMA,
        ],
    )(src, mask2d, moff)


BN = 1000


def _tc1_body(outx_ref, w_ref, degp_ref, h_ref, dinv_ref, dsq_ref):
    d = degp_ref[...] + 1.0
    dv = jax.lax.rsqrt(d)
    dinv_ref[...] = dv
    dsq_ref[...] = 1.0 / d
    h_ref[...] = jnp.dot(outx_ref[...], w_ref[...],
                         preferred_element_type=jnp.float32)


def _tc1(outx, W1, degp):
    return pl.pallas_call(
        _tc1_body,
        grid=(N // BN,),
        in_specs=[
            pl.BlockSpec((BN, D), lambda i: (i, 0)),
            pl.BlockSpec((D, D), lambda i: (0, 0)),
            pl.BlockSpec((BN, 1), lambda i: (i, 0)),
        ],
        out_specs=[
            pl.BlockSpec((BN, D), lambda i: (i, 0)),
            pl.BlockSpec((BN, 1), lambda i: (i, 0)),
            pl.BlockSpec((BN, 1), lambda i: (i, 0)),
        ],
        out_shape=[
            jax.ShapeDtypeStruct((N, D), jnp.float32),
            jax.ShapeDtypeStruct((N, 1), jnp.float32),
            jax.ShapeDtypeStruct((N, 1), jnp.float32),
        ],
    )(outx, W1, degp)


def _tc_combine_body(p_ref, h_ref, dsq_ref, b_ref, w_ref, z_ref, hn_ref, *,
                     do_relu, do_mm):
    z = p_ref[0] + p_ref[1] + dsq_ref[...] * h_ref[...] + b_ref[...]
    z_ref[...] = z
    if do_mm:
        a = jnp.maximum(z, 0.0) if do_relu else z
        hn_ref[...] = jnp.dot(a, w_ref[...], preferred_element_type=jnp.float32)


def _tc_combine(p, h, dsq, b, W, do_relu, do_mm):
    body = functools.partial(_tc_combine_body, do_relu=do_relu, do_mm=do_mm)
    return pl.pallas_call(
        body,
        grid=(N // BN,),
        in_specs=[
            pl.BlockSpec((NC, BN, D), lambda i: (0, i, 0)),
            pl.BlockSpec((BN, D), lambda i: (i, 0)),
            pl.BlockSpec((BN, 1), lambda i: (i, 0)),
            pl.BlockSpec((1, D), lambda i: (0, 0)),
            pl.BlockSpec((D, D), lambda i: (0, 0)),
        ],
        out_specs=[
            pl.BlockSpec((BN, D), lambda i: (i, 0)),
            pl.BlockSpec((BN, D), lambda i: (i, 0)),
        ],
        out_shape=[
            jax.ShapeDtypeStruct((N, D), jnp.float32),
            jax.ShapeDtypeStruct((N, D), jnp.float32),
        ],
    )(p, h, dsq, b, W)


def _tc_wf_body(a_ref, b_ref, o_ref):
    o_ref[...] = lax.dot_general(a_ref[...], b_ref[...],
                                 (((0,), (0,)), ((), ())),
                                 preferred_element_type=jnp.float32)


def _tc_wf(e2d_W, dec_W1):
    return pl.pallas_call(
        _tc_wf_body,
        out_shape=jax.ShapeDtypeStruct((D, D), jnp.float32),
    )(e2d_W, dec_W1)


def _tc_loss_body(xm_ref, dxm_ref, o_ref, *, n_mask):
    i = pl.program_id(0)

    @pl.when(i == 0)
    def _():
        o_ref[...] = jnp.zeros_like(o_ref)

    x = xm_ref[...]
    y = dxm_ref[...]
    nx = jnp.maximum(jnp.sqrt(jnp.sum(x * x, axis=1, keepdims=True)), 1e-12)
    ny = jnp.maximum(jnp.sqrt(jnp.sum(y * y, axis=1, keepdims=True)), 1e-12)
    dot = jnp.sum(x * y, axis=1, keepdims=True)
    r = 1.0 - dot / (nx * ny)
    o_ref[...] += jnp.sum(r * r).reshape(1, 1) * (1.0 / n_mask)


def _tc_loss(xm, dxm):
    n_mask = xm.shape[0]
    bm = 1000
    body = functools.partial(_tc_loss_body, n_mask=n_mask)
    return pl.pallas_call(
        body,
        grid=(n_mask // bm,),
        in_specs=[
            pl.BlockSpec((bm, D), lambda i: (i, 0)),
            pl.BlockSpec((bm, D), lambda i: (i, 0)),
        ],
        out_specs=pl.BlockSpec((1, 1), lambda i: (0, 0)),
        out_shape=jax.ShapeDtypeStruct((1, 1), jnp.float32),
    )(xm, dxm)


def _pad_reshape16(idx):
    n = idx.shape[0]
    nb = (n + 15) // 16
    pad = nb * 16 - n
    if pad:
        idx = jnp.concatenate([idx, idx[:pad]])
    return idx.reshape(nb, 16)


def _overlap_tail16(idx):
    import numpy as np
    n = idx.shape[0]
    nb = (n + 15) // 16
    offs = np.minimum(np.arange(nb) * 16, n - 16).astype(np.int32)
    rows = [idx[o:o + 16] for o in offs]
    blocks = jnp.stack(rows)
    offs2d = jnp.asarray(np.broadcast_to(offs[:, None], (nb, 16)).copy())
    return blocks, offs2d


def kernel(x, edge_index, edge_weight, mask_nodes, token_nodes, noise_nodes,
           noise_src, mask_token, enc_W1, enc_b1, enc_W2, enc_b2, e2d_W,
           dec_W1, dec_b1, dec_W2, dec_b2):
    row = edge_index[0].astype(jnp.int32)
    col = edge_index[1].astype(jnp.int32)
    ew = edge_weight
    col_deg = col.reshape(NS, E // NS // 16, 16)
    ew_deg = ew.reshape(NS, E // NS // 16, 16)
    row_w = row.reshape(NW, EPT)
    col_w = col.reshape(NW, EPT)
    ew_w = ew.reshape(NW, EPT)
    col_b = col.reshape(NW, NBLK, BLK)
    zeros640 = jnp.zeros((640, D), jnp.float32)

    tok2d = _pad_reshape16(token_nodes)
    noise2d = _pad_reshape16(noise_nodes)
    nsrc2d = _pad_reshape16(noise_src)
    mask2d, moff = _overlap_tail16(mask_nodes)
    n_mask = mask_nodes.shape[0]

    outx, xm, degp = _sc_pre(x, tok2d, noise2d, nsrc2d, mask2d, moff,
                             mask_token, col_deg, ew_deg, n_mask)
    degp_n = degp[:N].reshape(N, 1)

    h1, dinv, dsq = _tc1(outx, enc_W1, degp_n)
    norm = _sc_norm(row_w, col_w, ew_w, dinv.reshape(N))

    bias = lambda b: b.reshape(1, D)
    wf = _tc_wf(e2d_W, dec_W1)

    p1 = _sc_msg(h1, row_w, col_b, norm, zeros640)
    _, h2 = _tc_combine(p1, h1, dsq, bias(enc_b1), enc_W2, True, True)
    p2 = _sc_msg(h2, row_w, col_b, norm, zeros640)
    enc_x, h3 = _tc_combine(p2, h2, dsq, bias(enc_b2), wf, False, True)
    p3 = _sc_msg(h3, row_w, col_b, norm, zeros640)
    _, h4 = _tc_combine(p3, h3, dsq, bias(dec_b1), dec_W2, True, True)
    p4 = _sc_msg(h4, row_w, col_b, norm, zeros640)
    dec_x, _ = _tc_combine(p4, h4, dsq, bias(dec_b2), dec_W2, False, False)

    dxm = _sc_gather(dec_x, mask2d, moff, n_mask)
    loss = _tc_loss(xm, dxm)[0, 0]
    return (enc_x, loss)
